# Initial kernel scaffold; baseline (speedup 1.0000x reference)
import jax
import jax.numpy as jnp
from jax.experimental import pallas as pl


def kernel(target_features, feature_dist_graph, rij_dist_pairs, b_scope, start_end_env, l_scope, scope_update, scope_update_lig, W_emb, b_emb, W_dist, b_dist, W_gat, a_gat):
    raise NotImplementedError("write your pallas kernel here")


# speedup vs baseline: 5.8005x; 5.8005x over previous
import functools
import jax
import jax.numpy as jnp
import numpy as np
from jax import lax
from jax.experimental import pallas as pl
from jax.experimental.pallas import tpu as pltpu
from jax.experimental.pallas import tpu_sc as plsc

H = 32
K = 30
NEG = -1e9


_NC, _NS = 2, 16
_NW = _NC * _NS
_CHUNK = 240


def _pad_len(m):
  per_round = _NW * _CHUNK
  return ((m + per_round - 1) // per_round) * per_round


@functools.partial(jax.jit, static_argnames=("d",))
def _sc_gather(table, idx, d):
  mp = idx.shape[0]
  per_tile = mp // _NW
  nch = per_tile // _CHUNK
  dt = table.dtype
  mesh = plsc.VectorSubcoreMesh(core_axis_name="c", subcore_axis_name="s")

  @functools.partial(
      pl.kernel,
      mesh=mesh,
      compiler_params=pltpu.CompilerParams(use_tc_tiling_on_sc=False),
      out_type=jax.ShapeDtypeStruct((mp, d), dt),
      scratch_types=[
          pltpu.VMEM((_CHUNK,), jnp.int32),
          pltpu.VMEM((_CHUNK, d), dt),
          pltpu.SemaphoreType.DMA,
      ],
  )
  def k(table_hbm, idx_hbm, out_hbm, idx_v, rows_v, sem):
    wid = lax.axis_index("s") * _NC + lax.axis_index("c")
    base = wid * per_tile

    def body(g, carry):
      off = base + g * _CHUNK
      pltpu.sync_copy(idx_hbm.at[pl.ds(off, _CHUNK)], idx_v)
      pltpu.async_copy(table_hbm.at[idx_v], rows_v, sem).wait()
      pltpu.sync_copy(rows_v, out_hbm.at[pl.ds(off, _CHUNK)])
      return carry

    lax.fori_loop(0, nch, body, 0)

  return k(table, idx)


def _pad_idx(idx, mp):
  return jnp.concatenate(
      [idx, jnp.zeros((mp - idx.shape[0],), jnp.int32)])


def _k_ei1(tf, v1, c1):
  n = tf.shape[0]
  blk = 1000

  def body(tf_ref, v_ref, c_ref, o_ref):
    r = jnp.dot(tf_ref[...], v_ref[...],
                preferred_element_type=jnp.float32) + c_ref[0, 0]
    o_ref[...] = jnp.broadcast_to(r, (blk, 8))

  return pl.pallas_call(
      body,
      grid=(n // blk,),
      in_specs=[
          pl.BlockSpec((blk, 8), lambda i: (i, 0)),
          pl.BlockSpec((8, 1), lambda i: (0, 0)),
          pl.BlockSpec((1, 1), lambda i: (0, 0)),
      ],
      out_specs=pl.BlockSpec((blk, 8), lambda i: (i, 0)),
      out_shape=jax.ShapeDtypeStruct((n, 8), jnp.float32),
  )(tf, v1, c1)


def _k_gm(fdg, rij, wc8, wc_last, bc):
  p = fdg.shape[0]
  blk = 6000

  def body(f_ref, r_ref, w_ref, wl_ref, b_ref, o_ref):
    o_ref[...] = (
        jnp.dot(f_ref[...], w_ref[...], preferred_element_type=jnp.float32)
        + r_ref[...] * wl_ref[...] + b_ref[...])

  return pl.pallas_call(
      body,
      grid=(p // blk,),
      in_specs=[
          pl.BlockSpec((blk, 8), lambda i: (i, 0)),
          pl.BlockSpec((blk, 1), lambda i: (i, 0)),
          pl.BlockSpec((8, H), lambda i: (0, 0)),
          pl.BlockSpec((1, H), lambda i: (0, 0)),
          pl.BlockSpec((1, H), lambda i: (0, 0)),
      ],
      out_specs=pl.BlockSpec((blk, H), lambda i: (i, 0)),
      out_shape=jax.ShapeDtypeStruct((p, H), jnp.float32),
  )(fdg, rij, wc8, wc_last, bc)


def _k_att(hjr, eig, bsc, a2m, em, sm):
  n = hjr.shape[0]
  blk = 784

  def body(h_ref, e_ref, b_ref, a2_ref, em_ref, s_ref, o_ref):
    hj = h_ref[...]
    ej = jnp.dot(hj, a2_ref[...], preferred_element_type=jnp.float32)
    e = e_ref[...] + ej
    e = jnp.where(e >= 0, e, 0.2 * e)
    mask = b_ref[...] > 0
    e = jnp.where(mask, e, NEG)
    m = jnp.max(e, axis=1, keepdims=True)
    pexp = jnp.exp(e - m)
    s = jnp.sum(pexp, axis=1, keepdims=True)
    alpha = pexp / s
    alpha = jnp.where(mask, alpha, 0.0)
    ar = jnp.dot(alpha, em_ref[...], preferred_element_type=jnp.float32)
    o = jnp.dot(ar * hj, s_ref[...], preferred_element_type=jnp.float32)
    o_ref[...] = jnp.where(o > 0, o, jnp.exp(jnp.minimum(o, 0.0)) - 1.0)

  return pl.pallas_call(
      body,
      grid=(n // blk,),
      in_specs=[
          pl.BlockSpec((blk, K * H), lambda i: (i, 0)),
          pl.BlockSpec((blk, K), lambda i: (i, 0)),
          pl.BlockSpec((blk, K), lambda i: (i, 0)),
          pl.BlockSpec((K * H, K), lambda i: (0, 0)),
          pl.BlockSpec((K, K * H), lambda i: (0, 0)),
          pl.BlockSpec((K * H, H), lambda i: (0, 0)),
      ],
      out_specs=pl.BlockSpec((blk, H), lambda i: (i, 0)),
      out_shape=jax.ShapeDtypeStruct((n, H), jnp.float32),
  )(hjr, eig, bsc, a2m, em, sm)


def _k_gt2(out1, wg, a1):
  n = out1.shape[0]
  blk = 1000

  def body(x_ref, w_ref, a_ref, g_ref, e_ref):
    g = jnp.dot(x_ref[...], w_ref[...], preferred_element_type=jnp.float32)
    g_ref[...] = g
    e_ref[...] = jnp.broadcast_to(
        jnp.dot(g, a_ref[...], preferred_element_type=jnp.float32), (blk, 8))

  return pl.pallas_call(
      body,
      grid=(n // blk,),
      in_specs=[
          pl.BlockSpec((blk, H), lambda i: (i, 0)),
          pl.BlockSpec((H, H), lambda i: (0, 0)),
          pl.BlockSpec((H, 1), lambda i: (0, 0)),
      ],
      out_specs=[
          pl.BlockSpec((blk, H), lambda i: (i, 0)),
          pl.BlockSpec((blk, 8), lambda i: (i, 0)),
      ],
      out_shape=[
          jax.ShapeDtypeStruct((n, H), jnp.float32),
          jax.ShapeDtypeStruct((n, 8), jnp.float32),
      ],
  )(out1, wg, a1)


def _k_molsum(rows, s50):
  b = rows.shape[0]

  def body(r_ref, s_ref, o_ref):
    o_ref[...] = jnp.dot(r_ref[...], s_ref[...],
                         preferred_element_type=jnp.float32)

  return pl.pallas_call(
      body,
      in_specs=[
          pl.BlockSpec((b, 50 * H), lambda: (0, 0)),
          pl.BlockSpec((50 * H, H), lambda: (0, 0)),
      ],
      out_specs=pl.BlockSpec((b, H), lambda: (0, 0)),
      out_shape=jax.ShapeDtypeStruct((b, H), jnp.float32),
  )(rows, s50)


def kernel(target_features, feature_dist_graph, rij_dist_pairs, b_scope,
           start_end_env, l_scope, scope_update, scope_update_lig, W_emb,
           b_emb, W_dist, b_dist, W_gat, a_gat):
  n = target_features.shape[0]
  p = feature_dist_graph.shape[0]
  b = l_scope.shape[0]
  m = n * K
  mp = _pad_len(m)

  a1 = a_gat[:H]
  a2 = a_gat[H:]
  wga1 = (W_gat @ a1).reshape(H, 1)
  v1 = (W_emb @ wga1).reshape(8, 1)
  c1 = (b_emb @ wga1).reshape(1, 1)
  wc = W_dist @ W_gat
  bc = (b_dist @ W_gat).reshape(1, H)

  eye_h = jnp.eye(H, dtype=jnp.float32)
  a2m = jnp.kron(jnp.eye(K, dtype=jnp.float32), a2.reshape(H, 1))
  em = jnp.kron(jnp.eye(K, dtype=jnp.float32),
                jnp.ones((1, H), jnp.float32))
  sm = jnp.kron(jnp.ones((K, 1), jnp.float32), eye_h)
  s50 = jnp.kron(jnp.ones((50, 1), jnp.float32), eye_h)

  bm1 = _pad_idx(jnp.maximum(b_scope.reshape(-1) - 1, 0), mp)
  see_flat = _pad_idx(start_end_env.reshape(-1), mp)
  mask2d = b_scope

  npad = mp // K
  nblk_pad = npad - n

  def pad_nodes(x):
    return jnp.concatenate(
        [x, jnp.zeros((nblk_pad,) + x.shape[1:], x.dtype)])

  ei1 = _k_ei1(target_features, v1, c1)
  ei1t = jnp.concatenate([jnp.zeros((1, 8), jnp.float32), ei1])
  gm = _k_gm(feature_dist_graph, rij_dist_pairs.reshape(p, 1),
             wc[:8], wc[8:9], bc)

  hj1 = _sc_gather(gm, bm1, H)
  eig1 = _sc_gather(ei1t, see_flat, 8)

  hjr1 = hj1.reshape(npad, K * H)
  eig1r = eig1[:, 0].reshape(npad, K)
  bscp = pad_nodes(mask2d)
  out1 = _k_att(hjr1, eig1r, bscp, a2m, em, sm)[:n]

  gt2, ei2 = _k_gt2(out1, W_gat, a1.reshape(H, 1))
  gt2p = jnp.concatenate([jnp.zeros((1, H), jnp.float32), gt2])
  ei2t = jnp.concatenate([jnp.zeros((1, 8), jnp.float32), ei2])

  su8 = jnp.broadcast_to(scope_update.reshape(p, 1), (p, 8))
  c8 = _sc_gather(jnp.asarray(su8, jnp.int32), bm1, 8)
  cidx = c8[:, 0]
  hj2 = _sc_gather(gt2p, cidx, H)
  eig2 = _sc_gather(ei2t, see_flat, 8)

  hjr2 = hj2.reshape(npad, K * H)
  eig2r = eig2[:, 0].reshape(npad, K)
  out2 = _k_att(hjr2, eig2r, bscp, a2m, em, sm)[:n]

  tfp3 = jnp.concatenate([jnp.zeros((1, H), jnp.float32), out2])
  lm = b * 50
  lmp = _pad_len(lm)
  l_flat = _pad_idx(l_scope.reshape(-1), lmp)
  rows = _sc_gather(tfp3, l_flat, H)[:lm].reshape(b, 50 * H)
  return _k_molsum(rows, s50)
